# baseline (device time: 9473 ns/iter reference)
import jax
import jax.numpy as jnp
from jax import lax
from jax.experimental import pallas as pl
from jax.experimental.pallas import tpu as pltpu

N_COLS_GLOBAL = 2048
N_CHUNKS = 4


def kernel(x):
    m, n = x.shape
    rows, cols = m // 128, 128
    m_c = m // N_CHUNKS
    rows_c = rows // N_CHUNKS

    def expand_to_column(compact):
        blocks = jnp.broadcast_to(compact[:, None, :], (rows_c, cols, cols))
        expanded = blocks.reshape(m_c, cols)
        ridx = lax.broadcasted_iota(jnp.int32, (m_c, cols), 0)
        cidx = lax.broadcasted_iota(jnp.int32, (m_c, cols), 1)
        picked = jnp.where(cidx == ridx % cols, expanded, 0.0)
        return jnp.sum(picked, axis=1, keepdims=True)

    def body(x_hbm, out_ref, xbuf, acc_ref, recv_ref, copy_sems, send_sems, recv_sems):
        my_x = lax.axis_index("x")
        my_y = lax.axis_index("y")
        nbr = (my_x, 1 - my_y)

        barrier = pltpu.get_barrier_semaphore()
        pl.semaphore_signal(
            barrier, inc=1, device_id=nbr, device_id_type=pl.DeviceIdType.MESH
        )

        def copy_in(c):
            return pltpu.make_async_copy(
                x_hbm.at[pl.ds(c * m_c, m_c), :], xbuf.at[c], copy_sems.at[c]
            )

        def exchange(c):
            return pltpu.make_async_remote_copy(
                src_ref=acc_ref.at[c],
                dst_ref=recv_ref.at[c],
                send_sem=send_sems.at[c],
                recv_sem=recv_sems.at[c],
                device_id=nbr,
                device_id_type=pl.DeviceIdType.MESH,
            )

        for c in range(N_CHUNKS):
            copy_in(c).start()

        for c in range(N_CHUNKS):
            copy_in(c).wait()
            x3 = xbuf[c].reshape(rows_c, cols, n)
            acc_ref[c] = jnp.sum(x3, axis=2)
            if c == 0:
                pl.semaphore_wait(barrier, 1)
            exchange(c).start()

        for c in range(N_CHUNKS):
            exchange(c).wait()
            total = (acc_ref[c] + recv_ref[c]) * (1.0 / N_COLS_GLOBAL)
            out_ref[pl.ds(c * m_c, m_c), :] = expand_to_column(total)

    return pl.pallas_call(
        body,
        out_shape=jax.ShapeDtypeStruct((m, 1), jnp.float32),
        in_specs=[pl.BlockSpec(memory_space=pltpu.MemorySpace.HBM)],
        out_specs=pl.BlockSpec(memory_space=pltpu.VMEM),
        scratch_shapes=[
            pltpu.VMEM((N_CHUNKS, m_c, n), jnp.float32),
            pltpu.VMEM((N_CHUNKS, rows_c, cols), jnp.float32),
            pltpu.VMEM((N_CHUNKS, rows_c, cols), jnp.float32),
            pltpu.SemaphoreType.DMA((N_CHUNKS,)),
            pltpu.SemaphoreType.DMA((N_CHUNKS,)),
            pltpu.SemaphoreType.DMA((N_CHUNKS,)),
        ],
        compiler_params=pltpu.CompilerParams(collective_id=0),
    )(x)


# device time: 8812 ns/iter; 1.0750x vs baseline; 1.0750x over previous
import jax
import jax.numpy as jnp
from jax import lax
from jax.experimental import pallas as pl
from jax.experimental.pallas import tpu as pltpu

N_COLS_GLOBAL = 2048


def kernel(x):
    m, n = x.shape
    rows, cols = m // 128, 128

    def body(x_ref, out_ref, acc_ref, recv_ref, send_sem, recv_sem):
        my_x = lax.axis_index("x")
        my_y = lax.axis_index("y")
        nbr = (my_x, 1 - my_y)

        barrier = pltpu.get_barrier_semaphore()
        pl.semaphore_signal(
            barrier, inc=1, device_id=nbr, device_id_type=pl.DeviceIdType.MESH
        )

        x3 = x_ref[...].reshape(rows, cols, n)
        acc_ref[...] = jnp.sum(x3, axis=2)

        pl.semaphore_wait(barrier, 1)

        rdma = pltpu.make_async_remote_copy(
            src_ref=acc_ref,
            dst_ref=recv_ref,
            send_sem=send_sem,
            recv_sem=recv_sem,
            device_id=nbr,
            device_id_type=pl.DeviceIdType.MESH,
        )
        rdma.start()
        rdma.wait()

        total = (acc_ref[...] + recv_ref[...]) * (1.0 / N_COLS_GLOBAL)
        blocks = jnp.broadcast_to(total[:, None, :], (rows, cols, cols))
        expanded = blocks.reshape(m, cols)
        ridx = lax.broadcasted_iota(jnp.int32, (m, cols), 0)
        cidx = lax.broadcasted_iota(jnp.int32, (m, cols), 1)
        maskf = (cidx == ridx % cols).astype(jnp.float32)
        out_ref[...] = jax.lax.dot(
            expanded * maskf,
            jnp.ones((cols, 1), jnp.float32),
            preferred_element_type=jnp.float32,
        )

    return pl.pallas_call(
        body,
        out_shape=jax.ShapeDtypeStruct((m, 1), jnp.float32),
        in_specs=[pl.BlockSpec(memory_space=pltpu.VMEM)],
        out_specs=pl.BlockSpec(memory_space=pltpu.VMEM),
        scratch_shapes=[
            pltpu.VMEM((rows, cols), jnp.float32),
            pltpu.VMEM((rows, cols), jnp.float32),
            pltpu.SemaphoreType.DMA,
            pltpu.SemaphoreType.DMA,
        ],
        compiler_params=pltpu.CompilerParams(collective_id=0),
    )(x)
